# Initial kernel scaffold; baseline (speedup 1.0000x reference)
import jax
import jax.numpy as jnp
from jax.experimental import pallas as pl


def kernel(x, tables):
    raise NotImplementedError("write your pallas kernel here")


# speedup vs baseline: 41.9806x; 41.9806x over previous
import functools

import jax
import jax.numpy as jnp
from jax import lax
from jax.experimental import pallas as pl
from jax.experimental.pallas import tpu as pltpu
from jax.experimental.pallas import tpu_sc as plsc

LANES = 16


def kernel(x, tables):
    B, L = x.shape
    D = tables.shape[2]
    info = plsc.get_sparse_core_info()
    NC, NS = info.num_cores, info.num_subcores
    NW = NC * NS
    BW = B // NW
    CB = 16
    NCH = BW // CB
    CHUNK = CB * L
    NV = CHUNK // LANES

    flat_tab = tables.reshape(L * D, D)
    x_flat = x.reshape(B * L).astype(jnp.int32)
    off = jnp.tile(jnp.arange(L, dtype=jnp.int32) * D, CB)

    mesh = plsc.VectorSubcoreMesh(core_axis_name="c", subcore_axis_name="s")

    @functools.partial(
        pl.kernel,
        mesh=mesh,
        compiler_params=pltpu.CompilerParams(use_tc_tiling_on_sc=False),
        out_type=jax.ShapeDtypeStruct((B * L, D), jnp.float32),
        scratch_types=[
            pltpu.VMEM((CHUNK,), jnp.int32),
            pltpu.VMEM((CHUNK,), jnp.int32),
            pltpu.VMEM((CHUNK, D), jnp.float32),
            pltpu.SemaphoreType.DMA,
        ],
    )
    def k(x_hbm, off_hbm, tab_hbm, out_hbm, idx_v, off_v, rows_v, sem):
        wid = lax.axis_index("s") * NC + lax.axis_index("c")
        base = wid * (BW * L)
        pltpu.sync_copy(off_hbm, off_v)

        def chunk_body(c, carry):
            start = base + c * CHUNK
            pltpu.sync_copy(x_hbm.at[pl.ds(start, CHUNK)], idx_v)

            def add_body(j, carry2):
                s = pl.ds(pl.multiple_of(j * LANES, LANES), LANES)
                idx_v[s] = idx_v[s] + off_v[s]
                return carry2

            lax.fori_loop(0, NV, add_body, 0)
            pltpu.async_copy(tab_hbm.at[idx_v], rows_v, sem).wait()
            pltpu.sync_copy(rows_v, out_hbm.at[pl.ds(start, CHUNK)])
            return carry

        lax.fori_loop(0, NCH, chunk_body, 0)

    out = k(x_flat, off, flat_tab)
    return out.reshape(B, L * D)
